# baseline (speedup 1.0000x reference)
import functools

import jax
import jax.numpy as jnp
from jax import lax
from jax.experimental import pallas as pl
from jax.experimental.pallas import tpu as pltpu
from jax.experimental.pallas import tpu_sc as plsc

N_USERS = 1000000
N_ITEMS = 100000
N_FEATS = 100
K = 32
B = 4096

NC = 2
NS = 16
NW = NC * NS
BPW = B // NW

B_TILE = 512
D = K + 1
VF = (2 + N_FEATS) * K


def _sc_gather_body(u_hbm, i_hbm, ut_hbm, it_hbm, out_u, out_i,
                    uidx_v, urows_v, iidx_v, irows_v, sem_u, sem_i):
  wid = lax.axis_index("s") * NC + lax.axis_index("c")
  base = wid * BPW
  pltpu.sync_copy(u_hbm.at[pl.ds(base, BPW)], uidx_v)
  pltpu.sync_copy(i_hbm.at[pl.ds(base, BPW)], iidx_v)

  def fire(g, carry):
    uch = uidx_v[pl.ds(g * 16, 16)]
    ich = iidx_v[pl.ds(g * 16, 16)]
    for t in range(16):
      pltpu.make_async_copy(ut_hbm.at[pl.ds(uch[t], 1)],
                            urows_v.at[pl.ds(g * 16 + t, 1)], sem_u).start()
      pltpu.make_async_copy(it_hbm.at[pl.ds(ich[t], 1)],
                            irows_v.at[pl.ds(g * 16 + t, 1)], sem_i).start()
    return carry

  lax.fori_loop(0, BPW // 16, fire, 0)
  pltpu.make_async_copy(ut_hbm.at[pl.ds(0, BPW)], urows_v, sem_u).wait()
  pltpu.make_async_copy(it_hbm.at[pl.ds(0, BPW)], irows_v, sem_i).wait()

  pltpu.sync_copy(urows_v, out_u.at[pl.ds(base, BPW)])
  pltpu.sync_copy(irows_v, out_i.at[pl.ds(base, BPW)])


@functools.cache
def _sc_gather():
  return pl.kernel(
      _sc_gather_body,
      mesh=plsc.VectorSubcoreMesh(core_axis_name="c", subcore_axis_name="s"),
      out_type=(
          jax.ShapeDtypeStruct((B, D), jnp.float32),
          jax.ShapeDtypeStruct((B, D), jnp.float32),
      ),
      scratch_types=[
          pltpu.VMEM((BPW,), jnp.int32),
          pltpu.VMEM((BPW, D), jnp.float32),
          pltpu.VMEM((BPW,), jnp.int32),
          pltpu.VMEM((BPW, D), jnp.float32),
          pltpu.SemaphoreType.DMA,
          pltpu.SemaphoreType.DMA,
      ],
  )


def _tc_body(w0_ref, feats_ref, uv_ref, iv_ref, ftab_ref, m_ref,
             s_ref, w_ref, vflat_ref):
  feats = feats_ref[...]
  uv = uv_ref[...]
  iv = iv_ref[...]
  uvk = uv[:, :K]
  ivk = iv[:, :K]
  ftab_k = ftab_ref[:, :K]
  ftab_w = ftab_ref[:, K:]

  vflat_ref[:, 0:K] = uvk
  vflat_ref[:, K:2 * K] = ivk
  vflat_ref[:, 2 * K:] = jnp.dot(feats, m_ref[...],
                                 preferred_element_type=jnp.float32)

  wfeat = feats * ftab_w.reshape(1, N_FEATS)
  w_ref[:, 0:1] = uv[:, K:]
  w_ref[:, 1:2] = iv[:, K:]
  w_ref[:, 2:] = wfeat

  sv = uvk + ivk + jnp.dot(feats, ftab_k, preferred_element_type=jnp.float32)
  sq = (uvk * uvk + ivk * ivk
        + jnp.dot(feats * feats, ftab_k * ftab_k,
                  preferred_element_type=jnp.float32))
  v_ = 0.5 * (jnp.sum(sv * sv, axis=1, keepdims=True)
              - jnp.sum(sq, axis=1, keepdims=True))
  w_sum = (uv[:, K:] + iv[:, K:]
           + jnp.sum(wfeat, axis=1, keepdims=True))
  s_ref[...] = w0_ref[0, 0] + w_sum + v_


def _tc_fm(w0, feats, uv, iv, ftab, m):
  grid = (B // B_TILE,)
  return pl.pallas_call(
      _tc_body,
      grid=grid,
      in_specs=[
          pl.BlockSpec(memory_space=pltpu.SMEM),
          pl.BlockSpec((B_TILE, N_FEATS), lambda b: (b, 0)),
          pl.BlockSpec((B_TILE, D), lambda b: (b, 0)),
          pl.BlockSpec((B_TILE, D), lambda b: (b, 0)),
          pl.BlockSpec((N_FEATS, D), lambda b: (0, 0)),
          pl.BlockSpec((N_FEATS, N_FEATS * K), lambda b: (0, 0)),
      ],
      out_specs=[
          pl.BlockSpec((B_TILE, 1), lambda b: (b, 0)),
          pl.BlockSpec((B_TILE, 2 + N_FEATS), lambda b: (b, 0)),
          pl.BlockSpec((B_TILE, VF), lambda b: (b, 0)),
      ],
      out_shape=(
          jax.ShapeDtypeStruct((B, 1), jnp.float32),
          jax.ShapeDtypeStruct((B, 2 + N_FEATS), jnp.float32),
          jax.ShapeDtypeStruct((B, VF), jnp.float32),
      ),
  )(w0, feats, uv, iv, ftab, m)


def kernel(u, i, feats, user_table, item_table, feat_table, w0):
  u1 = u.reshape(B).astype(jnp.int32)
  i1 = i.reshape(B).astype(jnp.int32)

  uv, iv = _sc_gather()(u1, i1, user_table, item_table)

  ftab_k = feat_table[:, :K]
  m = (jnp.eye(N_FEATS, dtype=jnp.float32)[:, :, None]
       * ftab_k[None, :, :]).reshape(N_FEATS, N_FEATS * K)

  s, w, vflat = _tc_fm(w0.reshape(1, 1), feats, uv, iv, feat_table, m)
  return (s.reshape(B), w, vflat.reshape(B, 2 + N_FEATS, K))
